# baseline (speedup 1.0000x reference)
import functools

import jax
import jax.numpy as jnp
from jax import lax
from jax.experimental import pallas as pl
from jax.experimental.pallas import tpu as pltpu
from jax.experimental.pallas import tpu_sc as plsc

N = 1024
C = 256
D = 256
N_IT = 25
B = 8
NW = 32
RPW = (B * C) // NW


def _dot(a, b):
    return jax.lax.dot_general(
        a, b, (((1,), (0,)), ((), ())), preferred_element_type=jnp.float32
    )


def _centers_body(Wc_ref, b_ref, h_ref, cent_ref):
    f32 = jnp.float32
    h_b = h_ref[0]
    Wc = Wc_ref[...]
    bval = b_ref[0, 0]

    s_col = _dot(h_b, Wc) + bval

    s_row = jnp.transpose(s_col)
    ii = jax.lax.broadcasted_iota(jnp.int32, (N, N), 0)
    jj = jax.lax.broadcasted_iota(jnp.int32, (N, N), 1)

    M = (s_row < s_col) | ((s_row == s_col) & (jj < ii))
    rank_row = (N - 1.0) - jnp.sum(M.astype(f32), axis=0, keepdims=True)

    kcol = jax.lax.broadcasted_iota(jnp.int32, (C, 1), 0)
    targ = rank_row == (4.0 * kcol.astype(f32) + 2.0)
    cent0 = jnp.sum(jnp.where(targ, s_row, 0.0), axis=1, keepdims=True)

    m1 = jnp.sum(jnp.where(rank_row == 511.0, s_row, 0.0))
    m2 = jnp.sum(jnp.where(rank_row == 512.0, s_row, 0.0))
    med = (m1 + m2) * 0.5

    kk = jax.lax.broadcasted_iota(jnp.int32, (C, N), 0).astype(f32)
    jn = jax.lax.broadcasted_iota(jnp.int32, (C, N), 1).astype(f32)

    def assign_onehot(cent):
        d = jnp.abs(s_row - cent)
        dmin = jnp.min(d, axis=0, keepdims=True)
        amin = jnp.min(jnp.where(d == dmin, kk, float(C)), axis=0, keepdims=True)
        return amin == kk

    bf16 = jnp.bfloat16
    one_col = jnp.ones((N, 1), f32)
    rhs_su = jnp.concatenate([s_col, one_col], axis=1).astype(bf16)

    def step(cent):
        ohb = assign_onehot(cent).astype(bf16)
        sc = jax.lax.dot_general(
            ohb, rhs_su, (((1,), (0,)), ((), ())), preferred_element_type=f32
        )
        sums = sc[:, 0:1]
        counts = sc[:, 1:2]
        return jnp.where(counts > 0, sums / jnp.maximum(counts, 1.0), cent)

    def cond(carry):
        i, _, fixed = carry
        return jnp.logical_and(i < N_IT, jnp.logical_not(fixed))

    def body(carry):
        i, cent, _ = carry
        new = step(cent)
        fixed = jnp.sum((new != cent).astype(f32)) == 0.0
        return (i + 1, new, fixed)

    _, cent, _ = jax.lax.while_loop(
        cond, body, (jnp.int32(0), cent0, jnp.bool_(False))
    )

    oh = assign_onehot(cent)
    ohb = oh.astype(bf16)
    w_col = jax.nn.sigmoid(s_col)
    rhs_w = jnp.concatenate(
        [w_col, w_col * s_col, one_col], axis=1
    ).astype(bf16)
    snw = jax.lax.dot_general(
        ohb, rhs_w, (((1,), (0,)), ((), ())), preferred_element_type=f32
    )
    wsum = snw[:, 0:1]
    wnum = snw[:, 1:2]
    counts = snw[:, 2:3]
    wmean = wnum / jnp.where(wsum > 0, wsum, 1.0)
    diff = jnp.where(oh, jnp.abs(s_row - wmean), jnp.inf)
    dmin2 = jnp.min(diff, axis=1, keepdims=True)
    centers = jnp.min(jnp.where(diff == dmin2, jn, float(N)), axis=1, keepdims=True)

    dmed = jnp.abs(s_row - med)
    mm = jnp.min(dmed)
    iN = jax.lax.broadcasted_iota(jnp.int32, (1, N), 1).astype(f32)
    med_idx = jnp.min(jnp.where(dmed == mm, iN, float(N)))
    centers = jnp.where(counts > 0, centers, med_idx)

    cent_ref[...] = centers.astype(jnp.int32).reshape(1, C, 1)


_sc_mesh = plsc.VectorSubcoreMesh(core_axis_name="c", subcore_axis_name="s")


@functools.partial(
    pl.kernel,
    mesh=_sc_mesh,
    compiler_params=pltpu.CompilerParams(needs_layout_passes=False),
    out_type=[
        jax.ShapeDtypeStruct((B * C, C), jnp.float32),
        jax.ShapeDtypeStruct((B * C, D), jnp.float32),
    ],
    scratch_types=[
        pltpu.VMEM((RPW,), jnp.int32),
        pltpu.VMEM((RPW,), jnp.int32),
        pltpu.VMEM((C,), jnp.int32),
        pltpu.VMEM((RPW, D), jnp.float32),
        pltpu.VMEM((RPW, N), jnp.float32),
        pltpu.VMEM((RPW, C), jnp.float32),
        pltpu.SemaphoreType.DMA,
        pltpu.SemaphoreType.DMA,
    ],
)
def _sc_gather(cent_hbm, h_hbm, g_hbm, newg_hbm, newh_hbm,
               cidx_v, ridx_v, cb_v, rowsh_v, rowsg_v, outg_v, sem1, sem2):
    i32 = jnp.int32
    wid = lax.axis_index("s") * 2 + lax.axis_index("c")
    base = wid * RPW
    batch = base // C
    pltpu.sync_copy(cent_hbm.at[pl.ds(base, RPW)], cidx_v)
    pltpu.sync_copy(cent_hbm.at[pl.ds(batch * C, C)], cb_v)
    off = batch * N
    for t in range(RPW // 16):
        ridx_v[pl.ds(16 * t, 16)] = cidx_v[pl.ds(16 * t, 16)] + off
    cp1 = pltpu.async_copy(h_hbm.at[ridx_v], rowsh_v, sem1)
    cp2 = pltpu.async_copy(g_hbm.at[ridx_v], rowsg_v, sem2)
    cp1.wait()
    pltpu.sync_copy(rowsh_v, newh_hbm.at[pl.ds(base, RPW)])
    cp2.wait()

    lane = lax.iota(i32, 16)

    def row_body(r, carry):
        rvec = jnp.broadcast_to(r, (16,))
        for t in range(C // 16):
            cols = cb_v[pl.ds(16 * t, 16)]
            vals = plsc.load_gather(rowsg_v, [rvec, cols])
            plsc.store_scatter(outg_v, [rvec, lane + (16 * t)], vals)
        return carry

    lax.fori_loop(0, RPW, row_body, 0)
    pltpu.sync_copy(outg_v, newg_hbm.at[pl.ds(base, RPW)])


def kernel(g, h, W, b):
    Wc = W.reshape(D, 1)
    b2 = b.reshape(1, 1)
    centers = pl.pallas_call(
        _centers_body,
        grid=(B,),
        in_specs=[
            pl.BlockSpec((D, 1), lambda i: (0, 0)),
            pl.BlockSpec((1, 1), lambda i: (0, 0)),
            pl.BlockSpec((1, N, D), lambda i: (i, 0, 0)),
        ],
        out_specs=pl.BlockSpec((1, C, 1), lambda i: (i, 0, 0)),
        out_shape=jax.ShapeDtypeStruct((B, C, 1), jnp.int32),
    )(Wc, b2, h)
    new_g, new_h = _sc_gather(
        centers.reshape(B * C), h.reshape(B * N, D), g.reshape(B * N, N)
    )
    return (new_g.reshape(B, C, C), new_h.reshape(B, C, D))


# speedup vs baseline: 1.0514x; 1.0514x over previous
import functools

import jax
import jax.numpy as jnp
from jax import lax
from jax.experimental import pallas as pl
from jax.experimental.pallas import tpu as pltpu
from jax.experimental.pallas import tpu_sc as plsc

N = 1024
C = 256
D = 256
N_IT = 25
B = 8
NW = 32
RPW = (B * C) // NW


def _dot(a, b):
    return jax.lax.dot_general(
        a, b, (((1,), (0,)), ((), ())), preferred_element_type=jnp.float32
    )


def _centers_body(Wc_ref, b_ref, h_ref, cent_ref):
    f32 = jnp.float32
    h_b = h_ref[0]
    Wc = Wc_ref[...]
    bval = b_ref[0, 0]

    s_col = _dot(h_b, Wc) + bval

    s_row = jnp.transpose(s_col)
    ii = jax.lax.broadcasted_iota(jnp.int32, (N, N), 0)
    jj = jax.lax.broadcasted_iota(jnp.int32, (N, N), 1)

    M = (s_row < s_col) | ((s_row == s_col) & (jj < ii))
    rank_row = (N - 1.0) - jnp.sum(M.astype(f32), axis=0, keepdims=True)

    kcol = jax.lax.broadcasted_iota(jnp.int32, (C, 1), 0)
    targ = rank_row == (4.0 * kcol.astype(f32) + 2.0)
    cent0 = jnp.sum(jnp.where(targ, s_row, 0.0), axis=1, keepdims=True)

    m1 = jnp.sum(jnp.where(rank_row == 511.0, s_row, 0.0))
    m2 = jnp.sum(jnp.where(rank_row == 512.0, s_row, 0.0))
    med = (m1 + m2) * 0.5

    kk = jax.lax.broadcasted_iota(jnp.int32, (C, N), 0).astype(f32)
    jn = jax.lax.broadcasted_iota(jnp.int32, (C, N), 1).astype(f32)

    def assign_onehot(cent):
        d = jnp.abs(s_row - cent)
        dmin = jnp.min(d, axis=0, keepdims=True)
        amin = jnp.min(jnp.where(d == dmin, kk, float(C)), axis=0, keepdims=True)
        return amin == kk

    bf16 = jnp.bfloat16
    one_col = jnp.ones((N, 1), f32)
    rhs_su = jnp.concatenate([s_col, one_col], axis=1).astype(bf16)

    s_bf = s_col.astype(bf16)

    def step(cent):
        oh = assign_onehot(cent)
        ohb = oh.astype(bf16)
        sums = jax.lax.dot_general(
            ohb, s_bf, (((1,), (0,)), ((), ())), preferred_element_type=f32
        )
        counts = jnp.sum(oh.astype(f32), axis=1, keepdims=True)
        return jnp.where(counts > 0, sums / jnp.maximum(counts, 1.0), cent)

    def cond(carry):
        i, _, fixed = carry
        return jnp.logical_and(i < N_IT, jnp.logical_not(fixed))

    def body(carry):
        i, cent, _ = carry
        new = step(cent)
        fixed = jnp.sum((new != cent).astype(f32)) == 0.0
        return (i + 1, new, fixed)

    _, cent, _ = jax.lax.while_loop(
        cond, body, (jnp.int32(0), cent0, jnp.bool_(False))
    )

    oh = assign_onehot(cent)
    ohb = oh.astype(bf16)
    w_col = jax.nn.sigmoid(s_col)
    rhs_w = jnp.concatenate(
        [w_col, w_col * s_col, one_col], axis=1
    ).astype(bf16)
    snw = jax.lax.dot_general(
        ohb, rhs_w, (((1,), (0,)), ((), ())), preferred_element_type=f32
    )
    wsum = snw[:, 0:1]
    wnum = snw[:, 1:2]
    counts = snw[:, 2:3]
    wmean = wnum / jnp.where(wsum > 0, wsum, 1.0)
    diff = jnp.where(oh, jnp.abs(s_row - wmean), jnp.inf)
    dmin2 = jnp.min(diff, axis=1, keepdims=True)
    centers = jnp.min(jnp.where(diff == dmin2, jn, float(N)), axis=1, keepdims=True)

    dmed = jnp.abs(s_row - med)
    mm = jnp.min(dmed)
    iN = jax.lax.broadcasted_iota(jnp.int32, (1, N), 1).astype(f32)
    med_idx = jnp.min(jnp.where(dmed == mm, iN, float(N)))
    centers = jnp.where(counts > 0, centers, med_idx)

    cent_ref[...] = centers.astype(jnp.int32).reshape(1, C, 1)


_sc_mesh = plsc.VectorSubcoreMesh(core_axis_name="c", subcore_axis_name="s")


@functools.partial(
    pl.kernel,
    mesh=_sc_mesh,
    compiler_params=pltpu.CompilerParams(needs_layout_passes=False),
    out_type=[
        jax.ShapeDtypeStruct((B * C, C), jnp.float32),
        jax.ShapeDtypeStruct((B * C, D), jnp.float32),
    ],
    scratch_types=[
        pltpu.VMEM((RPW,), jnp.int32),
        pltpu.VMEM((RPW,), jnp.int32),
        pltpu.VMEM((C,), jnp.int32),
        pltpu.VMEM((RPW, D), jnp.float32),
        pltpu.VMEM((RPW, N), jnp.float32),
        pltpu.VMEM((RPW, C), jnp.float32),
        pltpu.SemaphoreType.DMA,
        pltpu.SemaphoreType.DMA,
    ],
)
def _sc_gather(cent_hbm, h_hbm, g_hbm, newg_hbm, newh_hbm,
               cidx_v, ridx_v, cb_v, rowsh_v, rowsg_v, outg_v, sem1, sem2):
    i32 = jnp.int32
    wid = lax.axis_index("s") * 2 + lax.axis_index("c")
    base = wid * RPW
    batch = base // C
    pltpu.sync_copy(cent_hbm.at[pl.ds(base, RPW)], cidx_v)
    pltpu.sync_copy(cent_hbm.at[pl.ds(batch * C, C)], cb_v)
    off = batch * N
    for t in range(RPW // 16):
        ridx_v[pl.ds(16 * t, 16)] = cidx_v[pl.ds(16 * t, 16)] + off
    cp1 = pltpu.async_copy(h_hbm.at[ridx_v], rowsh_v, sem1)
    cp2 = pltpu.async_copy(g_hbm.at[ridx_v], rowsg_v, sem2)
    cp1.wait()
    pltpu.sync_copy(rowsh_v, newh_hbm.at[pl.ds(base, RPW)])
    cp2.wait()

    lane = lax.iota(i32, 16)

    def row_body(r, carry):
        rvec = jnp.broadcast_to(r, (16,))
        for t in range(C // 16):
            cols = cb_v[pl.ds(16 * t, 16)]
            vals = plsc.load_gather(rowsg_v, [rvec, cols])
            plsc.store_scatter(outg_v, [rvec, lane + (16 * t)], vals)
        return carry

    lax.fori_loop(0, RPW, row_body, 0)
    pltpu.sync_copy(outg_v, newg_hbm.at[pl.ds(base, RPW)])


def kernel(g, h, W, b):
    Wc = W.reshape(D, 1)
    b2 = b.reshape(1, 1)
    centers = pl.pallas_call(
        _centers_body,
        grid=(B,),
        in_specs=[
            pl.BlockSpec((D, 1), lambda i: (0, 0)),
            pl.BlockSpec((1, 1), lambda i: (0, 0)),
            pl.BlockSpec((1, N, D), lambda i: (i, 0, 0)),
        ],
        out_specs=pl.BlockSpec((1, C, 1), lambda i: (i, 0, 0)),
        out_shape=jax.ShapeDtypeStruct((B, C, 1), jnp.int32),
    )(Wc, b2, h)
    new_g, new_h = _sc_gather(
        centers.reshape(B * C), h.reshape(B * N, D), g.reshape(B * N, N)
    )
    return (new_g.reshape(B, C, C), new_h.reshape(B, C, D))


# speedup vs baseline: 1.0770x; 1.0244x over previous
import functools

import jax
import jax.numpy as jnp
from jax import lax
from jax.experimental import pallas as pl
from jax.experimental.pallas import tpu as pltpu
from jax.experimental.pallas import tpu_sc as plsc

N = 1024
C = 256
D = 256
N_IT = 25
B = 8
NW = 32
RPW = (B * C) // NW


def _dot(a, b):
    return jax.lax.dot_general(
        a, b, (((1,), (0,)), ((), ())), preferred_element_type=jnp.float32
    )


def _centers_body(Wc_ref, b_ref, h_ref, cent_ref):
    f32 = jnp.float32
    h_b = h_ref[0]
    Wc = Wc_ref[...]
    bval = b_ref[0, 0]

    s_col = _dot(h_b, Wc) + bval

    s_row = jnp.transpose(s_col)
    ii = jax.lax.broadcasted_iota(jnp.int32, (N, N), 0)
    jj = jax.lax.broadcasted_iota(jnp.int32, (N, N), 1)

    M = (s_row < s_col) | ((s_row == s_col) & (jj < ii))
    rank_row = (N - 1.0) - jnp.sum(M.astype(f32), axis=0, keepdims=True)

    kcol = jax.lax.broadcasted_iota(jnp.int32, (C, 1), 0)
    targ = rank_row == (4.0 * kcol.astype(f32) + 2.0)
    cent0 = jnp.sum(jnp.where(targ, s_row, 0.0), axis=1, keepdims=True)

    m1 = jnp.sum(jnp.where(rank_row == 511.0, s_row, 0.0))
    m2 = jnp.sum(jnp.where(rank_row == 512.0, s_row, 0.0))
    med = (m1 + m2) * 0.5

    kk = jax.lax.broadcasted_iota(jnp.int32, (C, N), 0)

    def assign_onehot(cent):
        d = jnp.abs(s_row - cent)
        amin = jnp.argmin(d, axis=0)[None, :]
        return amin == kk

    def step(cent):
        ohf = assign_onehot(cent).astype(f32)
        sums = _dot(ohf, s_col)
        counts = jnp.sum(ohf, axis=1, keepdims=True)
        return jnp.where(counts > 0, sums / jnp.maximum(counts, 1.0), cent)

    def cond(carry):
        i, _, fixed = carry
        return jnp.logical_and(i < N_IT, jnp.logical_not(fixed))

    def body(carry):
        i, cent, _ = carry
        new = step(cent)
        fixed = jnp.sum((new != cent).astype(f32)) == 0.0
        return (i + 1, new, fixed)

    _, cent, _ = jax.lax.while_loop(
        cond, body, (jnp.int32(0), cent0, jnp.bool_(False))
    )

    oh = assign_onehot(cent)
    ohf = oh.astype(f32)
    w_col = jax.nn.sigmoid(s_col)
    wsum = _dot(ohf, w_col)
    wnum = _dot(ohf, w_col * s_col)
    wmean = wnum / jnp.where(wsum > 0, wsum, 1.0)
    diff = jnp.where(oh, jnp.abs(s_row - wmean), jnp.inf)
    centers = jnp.argmin(diff, axis=1)[:, None]
    counts = jnp.sum(ohf, axis=1, keepdims=True)

    dmed = jnp.abs(s_row - med)
    med_idx = jnp.argmin(dmed[0])
    centers = jnp.where(counts > 0, centers, med_idx)

    cent_ref[...] = centers.reshape(1, C, 1)


_sc_mesh = plsc.VectorSubcoreMesh(core_axis_name="c", subcore_axis_name="s")


@functools.partial(
    pl.kernel,
    mesh=_sc_mesh,
    compiler_params=pltpu.CompilerParams(needs_layout_passes=False),
    out_type=[
        jax.ShapeDtypeStruct((B * C, C), jnp.float32),
        jax.ShapeDtypeStruct((B * C, D), jnp.float32),
    ],
    scratch_types=[
        pltpu.VMEM((RPW,), jnp.int32),
        pltpu.VMEM((RPW,), jnp.int32),
        pltpu.VMEM((C,), jnp.int32),
        pltpu.VMEM((RPW, D), jnp.float32),
        pltpu.VMEM((RPW, N), jnp.float32),
        pltpu.VMEM((RPW, C), jnp.float32),
        pltpu.SemaphoreType.DMA,
        pltpu.SemaphoreType.DMA,
    ],
)
def _sc_gather(cent_hbm, h_hbm, g_hbm, newg_hbm, newh_hbm,
               cidx_v, ridx_v, cb_v, rowsh_v, rowsg_v, outg_v, sem1, sem2):
    i32 = jnp.int32
    wid = lax.axis_index("s") * 2 + lax.axis_index("c")
    base = wid * RPW
    batch = base // C
    pltpu.sync_copy(cent_hbm.at[pl.ds(base, RPW)], cidx_v)
    pltpu.sync_copy(cent_hbm.at[pl.ds(batch * C, C)], cb_v)
    off = batch * N
    for t in range(RPW // 16):
        ridx_v[pl.ds(16 * t, 16)] = cidx_v[pl.ds(16 * t, 16)] + off
    cp1 = pltpu.async_copy(h_hbm.at[ridx_v], rowsh_v, sem1)
    cp2 = pltpu.async_copy(g_hbm.at[ridx_v], rowsg_v, sem2)
    cp1.wait()
    pltpu.sync_copy(rowsh_v, newh_hbm.at[pl.ds(base, RPW)])
    cp2.wait()

    lane = lax.iota(i32, 16)

    def row_body(r, carry):
        rvec = jnp.broadcast_to(r, (16,))
        for t in range(C // 16):
            cols = cb_v[pl.ds(16 * t, 16)]
            vals = plsc.load_gather(rowsg_v, [rvec, cols])
            plsc.store_scatter(outg_v, [rvec, lane + (16 * t)], vals)
        return carry

    lax.fori_loop(0, RPW, row_body, 0)
    pltpu.sync_copy(outg_v, newg_hbm.at[pl.ds(base, RPW)])


def kernel(g, h, W, b):
    Wc = W.reshape(D, 1)
    b2 = b.reshape(1, 1)
    centers = pl.pallas_call(
        _centers_body,
        grid=(B,),
        in_specs=[
            pl.BlockSpec((D, 1), lambda i: (0, 0)),
            pl.BlockSpec((1, 1), lambda i: (0, 0)),
            pl.BlockSpec((1, N, D), lambda i: (i, 0, 0)),
        ],
        out_specs=pl.BlockSpec((1, C, 1), lambda i: (i, 0, 0)),
        out_shape=jax.ShapeDtypeStruct((B, C, 1), jnp.int32),
    )(Wc, b2, h)
    new_g, new_h = _sc_gather(
        centers.reshape(B * C), h.reshape(B * N, D), g.reshape(B * N, N)
    )
    return (new_g.reshape(B, C, C), new_h.reshape(B, C, D))


# speedup vs baseline: 1.1976x; 1.1120x over previous
import functools

import jax
import jax.numpy as jnp
from jax import lax
from jax.experimental import pallas as pl
from jax.experimental.pallas import tpu as pltpu
from jax.experimental.pallas import tpu_sc as plsc

N = 1024
C = 256
D = 256
N_IT = 25
B = 8
NW = 32
RPW = (B * C) // NW


def _dot(a, b):
    return jax.lax.dot_general(
        a, b, (((1,), (0,)), ((), ())), preferred_element_type=jnp.float32
    )


def _centers_body(Wc_ref, b_ref, h_ref, cent_ref):
    f32 = jnp.float32
    h_b = h_ref[0]
    Wc = Wc_ref[...]
    bval = b_ref[0, 0]

    s_col = _dot(h_b, Wc) + bval

    s_row = jnp.transpose(s_col)
    ii = jax.lax.broadcasted_iota(jnp.int32, (N, N), 0)
    jj = jax.lax.broadcasted_iota(jnp.int32, (N, N), 1)

    M = (s_row < s_col) | ((s_row == s_col) & (jj < ii))
    rank_row = (N - 1.0) - jnp.sum(M.astype(f32), axis=0, keepdims=True)

    kcol = jax.lax.broadcasted_iota(jnp.int32, (C, 1), 0)
    targ = rank_row == (4.0 * kcol.astype(f32) + 2.0)
    cent0 = jnp.sum(jnp.where(targ, s_row, 0.0), axis=1, keepdims=True)

    m1 = jnp.sum(jnp.where(rank_row == 511.0, s_row, 0.0))
    m2 = jnp.sum(jnp.where(rank_row == 512.0, s_row, 0.0))
    med = (m1 + m2) * 0.5

    kk = jax.lax.broadcasted_iota(jnp.int32, (C, N), 0)

    def assign_onehot(cent):
        d = jnp.abs(s_row - cent)
        amin = jnp.argmin(d, axis=0)[None, :]
        return amin == kk

    def step(cent):
        ohf = assign_onehot(cent).astype(f32)
        sums = _dot(ohf, s_col)
        counts = jnp.sum(ohf, axis=1, keepdims=True)
        return jnp.where(counts > 0, sums / jnp.maximum(counts, 1.0), cent)

    def cond(carry):
        i, _, fixed = carry
        return jnp.logical_and(i < N_IT, jnp.logical_not(fixed))

    def body(carry):
        i, cent, _ = carry
        new = step(cent)
        fixed = jnp.sum((new != cent).astype(f32)) == 0.0
        return (i + 1, new, fixed)

    _, cent, _ = jax.lax.while_loop(
        cond, body, (jnp.int32(0), cent0, jnp.bool_(False))
    )

    oh = assign_onehot(cent)
    ohf = oh.astype(f32)
    w_col = jax.nn.sigmoid(s_col)
    wsum = _dot(ohf, w_col)
    wnum = _dot(ohf, w_col * s_col)
    wmean = wnum / jnp.where(wsum > 0, wsum, 1.0)
    diff = jnp.where(oh, jnp.abs(s_row - wmean), jnp.inf)
    centers = jnp.argmin(diff, axis=1)[:, None]
    counts = jnp.sum(ohf, axis=1, keepdims=True)

    dmed = jnp.abs(s_row - med)
    med_idx = jnp.argmin(dmed[0])
    centers = jnp.where(counts > 0, centers, med_idx)

    cent_ref[...] = centers.reshape(1, C, 1)


_sc_mesh = plsc.VectorSubcoreMesh(core_axis_name="c", subcore_axis_name="s")


@functools.partial(
    pl.kernel,
    mesh=_sc_mesh,
    compiler_params=pltpu.CompilerParams(needs_layout_passes=False),
    out_type=[
        jax.ShapeDtypeStruct((B * C, C), jnp.float32),
        jax.ShapeDtypeStruct((B * C, D), jnp.float32),
    ],
    scratch_types=[
        pltpu.VMEM((RPW,), jnp.int32),
        pltpu.VMEM((RPW,), jnp.int32),
        pltpu.VMEM((C,), jnp.int32),
        pltpu.VMEM((RPW, D), jnp.float32),
        pltpu.VMEM((RPW, N), jnp.float32),
        pltpu.VMEM((RPW, C), jnp.float32),
        pltpu.SemaphoreType.DMA,
        pltpu.SemaphoreType.DMA,
    ],
)
def _sc_gather(cent_hbm, h_hbm, g_hbm, newg_hbm, newh_hbm,
               cidx_v, ridx_v, cb_v, rowsh_v, rowsg_v, outg_v, sem1, sem2):
    i32 = jnp.int32
    wid = lax.axis_index("s") * 2 + lax.axis_index("c")
    base = wid * RPW
    batch = base // C
    pltpu.sync_copy(cent_hbm.at[pl.ds(base, RPW)], cidx_v)
    pltpu.sync_copy(cent_hbm.at[pl.ds(batch * C, C)], cb_v)
    off = batch * N
    for t in range(RPW // 16):
        ridx_v[pl.ds(16 * t, 16)] = cidx_v[pl.ds(16 * t, 16)] + off
    cp1 = pltpu.async_copy(h_hbm.at[ridx_v], rowsh_v, sem1)
    cp2 = pltpu.async_copy(g_hbm.at[ridx_v], rowsg_v, sem2)
    cp1.wait()
    pltpu.sync_copy(rowsh_v, newh_hbm.at[pl.ds(base, RPW)])
    cp2.wait()

    lane = lax.iota(i32, 16)

    @plsc.parallel_loop(0, RPW, unroll=4)
    def row_body(r):
        rvec = jnp.broadcast_to(r, (16,))
        for t in range(C // 16):
            cols = cb_v[pl.ds(16 * t, 16)]
            vals = plsc.load_gather(rowsg_v, [rvec, cols])
            plsc.store_scatter(outg_v, [rvec, lane + (16 * t)], vals)

    pltpu.sync_copy(outg_v, newg_hbm.at[pl.ds(base, RPW)])


def kernel(g, h, W, b):
    Wc = W.reshape(D, 1)
    b2 = b.reshape(1, 1)
    centers = pl.pallas_call(
        _centers_body,
        grid=(B,),
        in_specs=[
            pl.BlockSpec((D, 1), lambda i: (0, 0)),
            pl.BlockSpec((1, 1), lambda i: (0, 0)),
            pl.BlockSpec((1, N, D), lambda i: (i, 0, 0)),
        ],
        out_specs=pl.BlockSpec((1, C, 1), lambda i: (i, 0, 0)),
        out_shape=jax.ShapeDtypeStruct((B, C, 1), jnp.int32),
    )(Wc, b2, h)
    new_g, new_h = _sc_gather(
        centers.reshape(B * C), h.reshape(B * N, D), g.reshape(B * N, N)
    )
    return (new_g.reshape(B, C, C), new_h.reshape(B, C, D))


# speedup vs baseline: 1.2001x; 1.0020x over previous
import functools

import jax
import jax.numpy as jnp
from jax import lax
from jax.experimental import pallas as pl
from jax.experimental.pallas import tpu as pltpu
from jax.experimental.pallas import tpu_sc as plsc

N = 1024
C = 256
D = 256
N_IT = 25
B = 8
NW = 32
RPW = (B * C) // NW


def _dot(a, b):
    return jax.lax.dot_general(
        a, b, (((1,), (0,)), ((), ())), preferred_element_type=jnp.float32
    )


def _centers_body(Wc_ref, b_ref, h_ref, cent_ref):
    f32 = jnp.float32
    h_b = h_ref[0]
    Wc = Wc_ref[...]
    bval = b_ref[0, 0]

    s_col = _dot(h_b, Wc) + bval

    s_row = jnp.transpose(s_col)
    ii = jax.lax.broadcasted_iota(jnp.int32, (N, N), 0)
    jj = jax.lax.broadcasted_iota(jnp.int32, (N, N), 1)

    M = (s_row < s_col) | ((s_row == s_col) & (jj < ii))
    rank_row = (N - 1.0) - jnp.sum(M.astype(f32), axis=0, keepdims=True)

    kcol = jax.lax.broadcasted_iota(jnp.int32, (C, 1), 0)
    targ = rank_row == (4.0 * kcol.astype(f32) + 2.0)
    cent0 = jnp.sum(jnp.where(targ, s_row, 0.0), axis=1, keepdims=True)

    m1 = jnp.sum(jnp.where(rank_row == 511.0, s_row, 0.0))
    m2 = jnp.sum(jnp.where(rank_row == 512.0, s_row, 0.0))
    med = (m1 + m2) * 0.5

    kk = jax.lax.broadcasted_iota(jnp.int32, (C, N), 0)

    def assign_onehot(cent):
        d = jnp.abs(s_row - cent)
        amin = jnp.argmin(d, axis=0)[None, :]
        return amin == kk

    def step(cent):
        ohf = assign_onehot(cent).astype(f32)
        sums = _dot(ohf, s_col)
        counts = jnp.sum(ohf, axis=1, keepdims=True)
        return jnp.where(counts > 0, sums / jnp.maximum(counts, 1.0), cent)

    def cond(carry):
        i, _, fixed = carry
        return jnp.logical_and(i < N_IT, jnp.logical_not(fixed))

    def body(carry):
        i, cent, _ = carry
        new = step(cent)
        fixed = jnp.sum((new != cent).astype(f32)) == 0.0
        return (i + 1, new, fixed)

    _, cent, _ = jax.lax.while_loop(
        cond, body, (jnp.int32(0), cent0, jnp.bool_(False))
    )

    oh = assign_onehot(cent)
    ohf = oh.astype(f32)
    w_col = jax.nn.sigmoid(s_col)
    wsum = _dot(ohf, w_col)
    wnum = _dot(ohf, w_col * s_col)
    wmean = wnum / jnp.where(wsum > 0, wsum, 1.0)
    diff = jnp.where(oh, jnp.abs(s_row - wmean), jnp.inf)
    centers = jnp.argmin(diff, axis=1)[:, None]
    counts = jnp.sum(ohf, axis=1, keepdims=True)

    dmed = jnp.abs(s_row - med)
    med_idx = jnp.argmin(dmed[0])
    centers = jnp.where(counts > 0, centers, med_idx)

    cent_ref[...] = centers.reshape(1, C, 1)


_sc_mesh = plsc.VectorSubcoreMesh(core_axis_name="c", subcore_axis_name="s")


@functools.partial(
    pl.kernel,
    mesh=_sc_mesh,
    compiler_params=pltpu.CompilerParams(needs_layout_passes=False),
    out_type=[
        jax.ShapeDtypeStruct((B * C, C), jnp.float32),
        jax.ShapeDtypeStruct((B * C, D), jnp.float32),
    ],
    scratch_types=[
        pltpu.VMEM((RPW,), jnp.int32),
        pltpu.VMEM((RPW,), jnp.int32),
        pltpu.VMEM((C,), jnp.int32),
        pltpu.VMEM((RPW, D), jnp.float32),
        pltpu.VMEM((RPW, N), jnp.float32),
        pltpu.VMEM((RPW, C), jnp.float32),
        pltpu.SemaphoreType.DMA,
        pltpu.SemaphoreType.DMA,
    ],
)
def _sc_gather(cent_hbm, h_hbm, g_hbm, newg_hbm, newh_hbm,
               cidx_v, ridx_v, cb_v, rowsh_v, rowsg_v, outg_v, sem1, sem2):
    i32 = jnp.int32
    wid = lax.axis_index("s") * 2 + lax.axis_index("c")
    base = wid * RPW
    batch = base // C
    pltpu.sync_copy(cent_hbm.at[pl.ds(base, RPW)], cidx_v)
    pltpu.sync_copy(cent_hbm.at[pl.ds(batch * C, C)], cb_v)
    off = batch * N
    for t in range(RPW // 16):
        ridx_v[pl.ds(16 * t, 16)] = cidx_v[pl.ds(16 * t, 16)] + off
    H = RPW // 2
    cp1 = pltpu.async_copy(h_hbm.at[ridx_v], rowsh_v, sem1)
    cp2a = pltpu.async_copy(
        g_hbm.at[ridx_v.at[pl.ds(0, H)]], rowsg_v.at[pl.ds(0, H)], sem2
    )
    cp2b = pltpu.async_copy(
        g_hbm.at[ridx_v.at[pl.ds(H, H)]], rowsg_v.at[pl.ds(H, H)], sem2
    )
    cp1.wait()
    pltpu.sync_copy(rowsh_v, newh_hbm.at[pl.ds(base, RPW)])

    lane = lax.iota(i32, 16)

    def half(lo):
        @plsc.parallel_loop(lo, lo + H, unroll=4)
        def row_body(r):
            rvec = jnp.broadcast_to(r, (16,))
            for t in range(C // 16):
                cols = cb_v[pl.ds(16 * t, 16)]
                vals = plsc.load_gather(rowsg_v, [rvec, cols])
                plsc.store_scatter(outg_v, [rvec, lane + (16 * t)], vals)

    cp2a.wait()
    half(0)
    cp2b.wait()
    half(H)
    pltpu.sync_copy(outg_v, newg_hbm.at[pl.ds(base, RPW)])


def kernel(g, h, W, b):
    Wc = W.reshape(D, 1)
    b2 = b.reshape(1, 1)
    centers = pl.pallas_call(
        _centers_body,
        grid=(B,),
        in_specs=[
            pl.BlockSpec((D, 1), lambda i: (0, 0)),
            pl.BlockSpec((1, 1), lambda i: (0, 0)),
            pl.BlockSpec((1, N, D), lambda i: (i, 0, 0)),
        ],
        out_specs=pl.BlockSpec((1, C, 1), lambda i: (i, 0, 0)),
        out_shape=jax.ShapeDtypeStruct((B, C, 1), jnp.int32),
    )(Wc, b2, h)
    new_g, new_h = _sc_gather(
        centers.reshape(B * C), h.reshape(B * N, D), g.reshape(B * N, N)
    )
    return (new_g.reshape(B, C, C), new_h.reshape(B, C, D))
